# baseline (speedup 1.0000x reference)
import functools

import jax
import jax.numpy as jnp
from jax.experimental import pallas as pl
from jax.experimental.pallas import tpu as pltpu

_TEMP = 0.07
_EPS = 1e-12
_ROWS = 32768


def _fused_kernel(x_ref, pr_ref, q0_ref, q1_ref, out_ref, acc_ref,
                  *, num_p, num_q, batch, nsteps):
    b = pl.program_id(0)
    j = pl.program_id(1)

    @pl.when((b == 0) & (j == 0))
    def _init():
        acc_ref[...] = jnp.zeros_like(acc_ref)

    xb = x_ref[0]
    pr = pr_ref[:, 0, 0, 0, :]
    sim = jax.lax.dot_general(pr, xb, (((1,), (1,)), ((), ())),
                              preferred_element_type=jnp.float32)
    mx = jnp.max(sim, axis=0, keepdims=True)
    onehot = (sim == mx).astype(jnp.float32)
    ones = jnp.ones((sim.shape[1], 1), jnp.float32)
    cnt = jax.lax.dot_general(onehot, ones, (((1,), (0,)), ((), ())),
                              preferred_element_type=jnp.float32)
    for bb in range(batch):
        @pl.when(b == bb)
        def _acc(bb=bb):
            acc_ref[bb] += cnt

    @pl.when((b == batch - 1) & (j == nsteps - 1))
    def _loss():
        q0 = q0_ref[...]
        q1 = q1_ref[...]
        q0n = q0 / jnp.maximum(
            jnp.sqrt(jnp.sum(q0 * q0, axis=1, keepdims=True)), _EPS)
        q1n = q1 / jnp.maximum(
            jnp.sqrt(jnp.sum(q1 * q1, axis=1, keepdims=True)), _EPS)
        total = jnp.zeros((1, 1), jnp.float32)
        for bb in range(batch):
            cntb = acc_ref[bb]
            pool = jax.lax.dot_general(cntb, pr_ref[:, 0, 0, 0, :],
                                       (((0,), (0,)), ((), ())),
                                       preferred_element_type=jnp.float32)
            n = pool / jnp.maximum(
                jnp.sqrt(jnp.sum(pool * pool, axis=1, keepdims=True)), _EPS)
            s_neg = jax.lax.dot_general(n, q0n, (((1,), (1,)), ((), ())),
                                        preferred_element_type=jnp.float32)
            logit_neg = s_neg / _TEMP
            m = jnp.max(logit_neg, axis=1, keepdims=True)
            l_neg = jnp.sum(jnp.exp(logit_neg - m), axis=1, keepdims=True)
            s_pos = jax.lax.dot_general(n, q1n, (((1,), (1,)), ((), ())),
                                        preferred_element_type=jnp.float32)
            logit_pos = s_pos / _TEMP - m
            elp = jnp.exp(logit_pos)
            terms = -(logit_pos - jnp.log(jnp.maximum(l_neg + elp, 1e-4)))
            loss_b = jnp.sum(terms, axis=1, keepdims=True) / num_q
            total = total + loss_b
        out_ref[...] = total / batch


def kernel(x, label, prototypes, queue0, queue1):
    del label
    B, C = x.shape[0], x.shape[1]
    S = x.shape[2] * x.shape[3] * x.shape[4]
    P = prototypes.shape[0]
    Q = queue0.shape[0]
    nsteps = S // _ROWS

    x3 = x.transpose(0, 2, 3, 4, 1).reshape(B, S, C)
    pr2 = prototypes.transpose(0, 2, 3, 4, 1)

    out = pl.pallas_call(
        functools.partial(_fused_kernel, num_p=P, num_q=Q, batch=B,
                          nsteps=nsteps),
        grid=(B, nsteps),
        in_specs=[
            pl.BlockSpec((1, _ROWS, C), lambda b, j: (b, j, 0)),
            pl.BlockSpec((P, 1, 1, 1, C), lambda b, j: (0, 0, 0, 0, 0)),
            pl.BlockSpec((Q, C), lambda b, j: (0, 0)),
            pl.BlockSpec((Q, C), lambda b, j: (0, 0)),
        ],
        out_specs=pl.BlockSpec((1, 1), lambda b, j: (0, 0)),
        out_shape=jax.ShapeDtypeStruct((1, 1), jnp.float32),
        scratch_shapes=[pltpu.VMEM((B, P, 1), jnp.float32)],
        compiler_params=pltpu.CompilerParams(
            dimension_semantics=("arbitrary", "arbitrary")),
    )(x3, pr2, queue0, queue1)
    return out.reshape(1)


# speedup vs baseline: 1.0264x; 1.0264x over previous
import functools

import jax
import jax.numpy as jnp
from jax.experimental import pallas as pl
from jax.experimental.pallas import tpu as pltpu

_TEMP = 0.07
_EPS = 1e-12
_ROWS = 16384


def _fused_kernel(x_ref, pr_ref, q0_ref, q1_ref, out_ref, acc_ref,
                  *, num_p, num_q, batch, nsteps):
    b = pl.program_id(0)
    j = pl.program_id(1)

    @pl.when((b == 0) & (j == 0))
    def _init():
        acc_ref[...] = jnp.zeros_like(acc_ref)

    xb = x_ref[0]
    pr = pr_ref[:, 0, 0, 0, :]
    sim = jax.lax.dot_general(pr, xb, (((1,), (1,)), ((), ())),
                              preferred_element_type=jnp.float32)
    mx = jnp.max(sim, axis=0, keepdims=True)
    onehot = (sim == mx).astype(jnp.float32)
    ones = jnp.ones((sim.shape[1], 1), jnp.float32)
    cnt = jax.lax.dot_general(onehot, ones, (((1,), (0,)), ((), ())),
                              preferred_element_type=jnp.float32)
    for bb in range(batch):
        @pl.when(b == bb)
        def _acc(bb=bb):
            acc_ref[bb] += cnt

    @pl.when((b == batch - 1) & (j == nsteps - 1))
    def _loss():
        q0 = q0_ref[...]
        q1 = q1_ref[...]
        q0n = q0 / jnp.maximum(
            jnp.sqrt(jnp.sum(q0 * q0, axis=1, keepdims=True)), _EPS)
        q1n = q1 / jnp.maximum(
            jnp.sqrt(jnp.sum(q1 * q1, axis=1, keepdims=True)), _EPS)
        total = jnp.zeros((1, 1), jnp.float32)
        for bb in range(batch):
            cntb = acc_ref[bb]
            pool = jax.lax.dot_general(cntb, pr_ref[:, 0, 0, 0, :],
                                       (((0,), (0,)), ((), ())),
                                       preferred_element_type=jnp.float32)
            n = pool / jnp.maximum(
                jnp.sqrt(jnp.sum(pool * pool, axis=1, keepdims=True)), _EPS)
            s_neg = jax.lax.dot_general(n, q0n, (((1,), (1,)), ((), ())),
                                        preferred_element_type=jnp.float32)
            logit_neg = s_neg / _TEMP
            m = jnp.max(logit_neg, axis=1, keepdims=True)
            l_neg = jnp.sum(jnp.exp(logit_neg - m), axis=1, keepdims=True)
            s_pos = jax.lax.dot_general(n, q1n, (((1,), (1,)), ((), ())),
                                        preferred_element_type=jnp.float32)
            logit_pos = s_pos / _TEMP - m
            elp = jnp.exp(logit_pos)
            terms = -(logit_pos - jnp.log(jnp.maximum(l_neg + elp, 1e-4)))
            loss_b = jnp.sum(terms, axis=1, keepdims=True) / num_q
            total = total + loss_b
        out_ref[...] = total / batch


def kernel(x, label, prototypes, queue0, queue1):
    del label
    B, C = x.shape[0], x.shape[1]
    S = x.shape[2] * x.shape[3] * x.shape[4]
    P = prototypes.shape[0]
    Q = queue0.shape[0]
    nsteps = S // _ROWS

    x3 = x.transpose(0, 2, 3, 4, 1).reshape(B, S, C)
    pr2 = prototypes.transpose(0, 2, 3, 4, 1)

    out = pl.pallas_call(
        functools.partial(_fused_kernel, num_p=P, num_q=Q, batch=B,
                          nsteps=nsteps),
        grid=(B, nsteps),
        in_specs=[
            pl.BlockSpec((1, _ROWS, C), lambda b, j: (b, j, 0)),
            pl.BlockSpec((P, 1, 1, 1, C), lambda b, j: (0, 0, 0, 0, 0)),
            pl.BlockSpec((Q, C), lambda b, j: (0, 0)),
            pl.BlockSpec((Q, C), lambda b, j: (0, 0)),
        ],
        out_specs=pl.BlockSpec((1, 1), lambda b, j: (0, 0)),
        out_shape=jax.ShapeDtypeStruct((1, 1), jnp.float32),
        scratch_shapes=[pltpu.VMEM((B, P, 1), jnp.float32)],
        compiler_params=pltpu.CompilerParams(
            dimension_semantics=("arbitrary", "arbitrary")),
    )(x3, pr2, queue0, queue1)
    return out.reshape(1)
